# baseline (speedup 1.0000x reference)
import functools

import jax
import jax.numpy as jnp
from jax import lax
from jax.experimental import pallas as pl
from jax.experimental.pallas import tpu as pltpu
from jax.experimental.pallas import tpu_sc as plsc

_INFO = plsc.get_sparse_core_info()
_NC = _INFO.num_cores
_NS = _INFO.num_subcores
_NW = _NC * _NS

_NBUF = 3
_CHUNK = 4
_TC_FRAC_NUM, _TC_FRAC_DEN = 1, 4


def _make_sc_gather(n_tok: int, n_sc: int, d: int, chunk: int, nbuf: int):
    b_per_w = n_sc // _NW
    n_chunks = b_per_w // chunk
    assert n_chunks >= 2 * nbuf
    mesh = plsc.VectorSubcoreMesh(core_axis_name="c", subcore_axis_name="s")

    @functools.partial(
        pl.kernel,
        mesh=mesh,
        out_type=jax.ShapeDtypeStruct((n_tok, d), jnp.float32),
        scratch_types=[
            pltpu.VMEM((n_chunks, chunk), jnp.int32),
        ] + [pltpu.VMEM((chunk, d), jnp.float32)] * nbuf
          + [pltpu.SemaphoreType.DMA] * (2 * nbuf),
    )
    def gather_kernel(table_hbm, idx_hbm, out_hbm, idx_v, *rest):
        bufs = rest[:nbuf]
        gsems = rest[nbuf:2 * nbuf]
        wsems = rest[2 * nbuf:]

        wid = lax.axis_index("s") * _NC + lax.axis_index("c")
        base = wid * b_per_w
        pltpu.sync_copy(idx_hbm.at[wid], idx_v)

        def out_rows(g):
            return out_hbm.at[pl.ds(base + g * chunk, chunk)]

        def start_gather(g, b):
            pltpu.async_copy(table_hbm.at[idx_v.at[g]], bufs[b], gsems[b])

        def visit(g, b, static):
            bn = (b + nbuf - 1) % nbuf
            pltpu.make_async_copy(
                table_hbm.at[idx_v.at[g]], bufs[b], gsems[b]).wait()

            def recycle_wait():
                pltpu.make_async_copy(
                    bufs[bn], out_rows(g - 1), wsems[bn]).wait()

            def next_gather():
                start_gather(g + nbuf - 1, bn)

            if static:
                if g >= 1 and g + nbuf - 1 < n_chunks:
                    recycle_wait()
                if g + nbuf - 1 < n_chunks:
                    next_gather()
            else:
                pl.when((g >= 1) & (g + nbuf - 1 < n_chunks))(recycle_wait)
                pl.when(g + nbuf - 1 < n_chunks)(next_gather)

            pltpu.async_copy(bufs[b], out_rows(g), wsems[b])

        for b in range(nbuf - 1):
            start_gather(b, b)

        n_full = (n_chunks // nbuf) * nbuf

        def ring_body(t, carry):
            for b in range(nbuf):
                visit(nbuf * t + b, b, static=False)
            return carry

        lax.fori_loop(0, n_full // nbuf, ring_body, 0)

        for g in range(n_full, n_chunks):
            visit(g, g % nbuf, static=True)

        for g in range(n_chunks - nbuf, n_chunks):
            b = g % nbuf
            pltpu.make_async_copy(bufs[b], out_rows(g), wsems[b]).wait()

    return gather_kernel


def _tc_row_copy(idx_ref, emb_ref, out_ref):
    out_ref[...] = emb_ref[...]


def _make_tc_gather(n_tc: int, v: int, d: int):
    grid_spec = pltpu.PrefetchScalarGridSpec(
        num_scalar_prefetch=1,
        grid=(n_tc,),
        in_specs=[pl.BlockSpec((1, 1, d),
                               lambda i, idx_ref: (idx_ref[i], 0, 0))],
        out_specs=pl.BlockSpec((1, 1, d), lambda i, idx_ref: (i, 0, 0)),
    )
    return pl.pallas_call(
        _tc_row_copy,
        grid_spec=grid_spec,
        out_shape=jax.ShapeDtypeStruct((n_tc, 1, d), jnp.float32),
    )


def kernel(idx, embedding):
    b, s = idx.shape
    v, d = embedding.shape
    n_tok = b * s
    n_tc = (n_tok * _TC_FRAC_NUM // _TC_FRAC_DEN) // _NW * _NW
    n_sc = n_tok - n_tc

    idx_flat = idx.reshape(n_tok).astype(jnp.int32)
    idx_sc = idx_flat[:n_sc].reshape(_NW, (n_sc // _NW) // _CHUNK, _CHUNK)
    idx_tc = idx_flat[n_sc:]

    sc_full = _make_sc_gather(n_tok, n_sc, d, _CHUNK, _NBUF)(embedding, idx_sc)
    tc_tail = _make_tc_gather(n_tc, v, d)(
        idx_tc, embedding.reshape(v, 1, d)).reshape(n_tc, d)
    out = lax.dynamic_update_slice(sc_full, tc_tail, (n_sc, 0))
    return out.reshape(b, s, d)


# speedup vs baseline: 3.3928x; 3.3928x over previous
import functools

import jax
import jax.numpy as jnp
from jax import lax
from jax.experimental import pallas as pl
from jax.experimental.pallas import tpu as pltpu
from jax.experimental.pallas import tpu_sc as plsc

_INFO = plsc.get_sparse_core_info()
_NC = _INFO.num_cores
_NS = _INFO.num_subcores
_NW = _NC * _NS

_NBUF = 3
_CHUNK = 4
_TC_FRAC_NUM, _TC_FRAC_DEN = 1, 4


def _make_sc_gather(n_tok: int, n_sc: int, d: int, chunk: int, nbuf: int):
    b_per_w = n_sc // _NW
    n_chunks = b_per_w // chunk
    assert n_chunks >= 2 * nbuf
    mesh = plsc.VectorSubcoreMesh(core_axis_name="c", subcore_axis_name="s")

    @functools.partial(
        pl.kernel,
        mesh=mesh,
        out_type=jax.ShapeDtypeStruct((n_tok, d), jnp.float32),
        scratch_types=[
            pltpu.VMEM((n_chunks, chunk), jnp.int32),
        ] + [pltpu.VMEM((chunk, d), jnp.float32)] * nbuf
          + [pltpu.SemaphoreType.DMA] * (2 * nbuf),
    )
    def gather_kernel(table_hbm, idx_hbm, out_hbm, idx_v, *rest):
        bufs = rest[:nbuf]
        gsems = rest[nbuf:2 * nbuf]
        wsems = rest[2 * nbuf:]

        wid = lax.axis_index("s") * _NC + lax.axis_index("c")
        base = wid * b_per_w
        pltpu.sync_copy(idx_hbm.at[wid], idx_v)

        def out_rows(g):
            return out_hbm.at[pl.ds(base + g * chunk, chunk)]

        def start_gather(g, b):
            pltpu.async_copy(table_hbm.at[idx_v.at[g]], bufs[b], gsems[b])

        def visit(g, b, static):
            bn = (b + nbuf - 1) % nbuf
            pltpu.make_async_copy(
                table_hbm.at[idx_v.at[g]], bufs[b], gsems[b]).wait()

            def recycle_wait():
                pltpu.make_async_copy(
                    bufs[bn], out_rows(g - 1), wsems[bn]).wait()

            def next_gather():
                start_gather(g + nbuf - 1, bn)

            if static:
                if g >= 1 and g + nbuf - 1 < n_chunks:
                    recycle_wait()
                if g + nbuf - 1 < n_chunks:
                    next_gather()
            else:
                pl.when((g >= 1) & (g + nbuf - 1 < n_chunks))(recycle_wait)
                pl.when(g + nbuf - 1 < n_chunks)(next_gather)

            pltpu.async_copy(bufs[b], out_rows(g), wsems[b])

        for b in range(nbuf - 1):
            start_gather(b, b)

        n_full = (n_chunks // nbuf) * nbuf

        def ring_body(t, carry):
            for b in range(nbuf):
                visit(nbuf * t + b, b, static=False)
            return carry

        lax.fori_loop(0, n_full // nbuf, ring_body, 0)

        for g in range(n_full, n_chunks):
            visit(g, g % nbuf, static=True)

        for g in range(n_chunks - nbuf, n_chunks):
            b = g % nbuf
            pltpu.make_async_copy(bufs[b], out_rows(g), wsems[b]).wait()

    return gather_kernel


_G = 8


def _tc_body(idx_ref, emb_hbm, out_ref, sem):
    i = pl.program_id(0)

    def copies():
        for j in range(_G):
            yield pltpu.make_async_copy(
                emb_hbm.at[pl.ds(idx_ref[i * _G + j], 1), :],
                out_ref.at[pl.ds(j, 1), :],
                sem)

    for cp in copies():
        cp.start()
    for cp in copies():
        cp.wait()


def _make_tc_gather(n_tc: int, v: int, d: int):
    grid_spec = pltpu.PrefetchScalarGridSpec(
        num_scalar_prefetch=1,
        grid=(n_tc // _G,),
        in_specs=[pl.BlockSpec(memory_space=pltpu.MemorySpace.HBM)],
        out_specs=pl.BlockSpec((_G, d), lambda i, idx_ref: (i, 0)),
        scratch_shapes=[pltpu.SemaphoreType.DMA],
    )
    return pl.pallas_call(
        _tc_body,
        grid_spec=grid_spec,
        out_shape=jax.ShapeDtypeStruct((n_tc, d), jnp.float32),
    )


def kernel(idx, embedding):
    b, s = idx.shape
    v, d = embedding.shape
    n_tok = b * s
    n_tc = (n_tok * _TC_FRAC_NUM // _TC_FRAC_DEN) // _NW * _NW
    n_sc = n_tok - n_tc

    idx_flat = idx.reshape(n_tok).astype(jnp.int32)
    idx_sc = idx_flat[:n_sc].reshape(_NW, (n_sc // _NW) // _CHUNK, _CHUNK)
    idx_tc = idx_flat[n_sc:]

    sc_full = _make_sc_gather(n_tok, n_sc, d, _CHUNK, _NBUF)(embedding, idx_sc)
    tc_tail = _make_tc_gather(n_tc, v, d)(idx_tc, embedding)
    out = lax.dynamic_update_slice(sc_full, tc_tail, (n_sc, 0))
    return out.reshape(b, s, d)


# speedup vs baseline: 5.4733x; 1.6132x over previous
import functools

import jax
import jax.numpy as jnp
from jax import lax
from jax.experimental import pallas as pl
from jax.experimental.pallas import tpu as pltpu
from jax.experimental.pallas import tpu_sc as plsc

_INFO = plsc.get_sparse_core_info()
_NC = _INFO.num_cores
_NS = _INFO.num_subcores
_NW = _NC * _NS

_NBUF = 3
_CHUNK = 4
_TC_FRAC_NUM, _TC_FRAC_DEN = 1, 4


def _make_sc_gather(n_tok: int, n_sc: int, d: int, chunk: int, nbuf: int):
    b_per_w = n_sc // _NW
    n_chunks = b_per_w // chunk
    assert n_chunks >= 2 * nbuf
    mesh = plsc.VectorSubcoreMesh(core_axis_name="c", subcore_axis_name="s")

    @functools.partial(
        pl.kernel,
        mesh=mesh,
        out_type=jax.ShapeDtypeStruct((n_tok, d), jnp.float32),
        scratch_types=[
            pltpu.VMEM((n_chunks, chunk), jnp.int32),
        ] + [pltpu.VMEM((chunk, d), jnp.float32)] * nbuf
          + [pltpu.SemaphoreType.DMA] * (2 * nbuf),
    )
    def gather_kernel(table_hbm, idx_hbm, out_hbm, idx_v, *rest):
        bufs = rest[:nbuf]
        gsems = rest[nbuf:2 * nbuf]
        wsems = rest[2 * nbuf:]

        wid = lax.axis_index("s") * _NC + lax.axis_index("c")
        base = wid * b_per_w
        pltpu.sync_copy(idx_hbm.at[wid], idx_v)

        def out_rows(g):
            return out_hbm.at[pl.ds(base + g * chunk, chunk)]

        def start_gather(g, b):
            pltpu.async_copy(table_hbm.at[idx_v.at[g]], bufs[b], gsems[b])

        def visit(g, b, static):
            bn = (b + nbuf - 1) % nbuf
            pltpu.make_async_copy(
                table_hbm.at[idx_v.at[g]], bufs[b], gsems[b]).wait()

            def recycle_wait():
                pltpu.make_async_copy(
                    bufs[bn], out_rows(g - 1), wsems[bn]).wait()

            def next_gather():
                start_gather(g + nbuf - 1, bn)

            if static:
                if g >= 1 and g + nbuf - 1 < n_chunks:
                    recycle_wait()
                if g + nbuf - 1 < n_chunks:
                    next_gather()
            else:
                pl.when((g >= 1) & (g + nbuf - 1 < n_chunks))(recycle_wait)
                pl.when(g + nbuf - 1 < n_chunks)(next_gather)

            pltpu.async_copy(bufs[b], out_rows(g), wsems[b])

        for b in range(nbuf - 1):
            start_gather(b, b)

        n_full = (n_chunks // nbuf) * nbuf

        def ring_body(t, carry):
            for b in range(nbuf):
                visit(nbuf * t + b, b, static=False)
            return carry

        lax.fori_loop(0, n_full // nbuf, ring_body, 0)

        for g in range(n_full, n_chunks):
            visit(g, g % nbuf, static=True)

        for g in range(n_chunks - nbuf, n_chunks):
            b = g % nbuf
            pltpu.make_async_copy(bufs[b], out_rows(g), wsems[b]).wait()

    return gather_kernel


_TG = 16
_TNBUF = 6


def _make_tc_gather(n_tc: int, v: int, d: int):
    n_chunks = n_tc // _TG
    assert n_chunks >= 2 * _TNBUF
    nbuf = _TNBUF

    def tc_body(idx_ref, emb_hbm, out_hbm, *rest):
        bufs = rest[:nbuf]
        gsems = rest[nbuf:2 * nbuf]
        wsems = rest[2 * nbuf:]

        def row_copy(g, j, b):
            return pltpu.make_async_copy(
                emb_hbm.at[pl.ds(idx_ref[g * _TG + j], 1), :],
                bufs[b].at[pl.ds(j, 1), :],
                gsems[b])

        def out_copy(g, b):
            return pltpu.make_async_copy(
                bufs[b], out_hbm.at[pl.ds(g * _TG, _TG)], wsems[b])

        def start_gather(g, b):
            for j in range(_TG):
                row_copy(g, j, b).start()

        def visit(g, b, static):
            bn = (b + nbuf - 1) % nbuf
            for j in range(_TG):
                row_copy(g, j, b).wait()

            def recycle_wait():
                out_copy(g - 1, bn).wait()

            def next_gather():
                start_gather(g + nbuf - 1, bn)

            if static:
                if g >= 1 and g + nbuf - 1 < n_chunks:
                    recycle_wait()
                if g + nbuf - 1 < n_chunks:
                    next_gather()
            else:
                pl.when((g >= 1) & (g + nbuf - 1 < n_chunks))(recycle_wait)
                pl.when(g + nbuf - 1 < n_chunks)(next_gather)

            out_copy(g, b).start()

        for b in range(nbuf - 1):
            start_gather(b, b)

        n_full = (n_chunks // nbuf) * nbuf

        def ring_body(t, carry):
            for b in range(nbuf):
                visit(nbuf * t + b, b, static=False)
            return carry

        lax.fori_loop(0, n_full // nbuf, ring_body, 0)

        for g in range(n_full, n_chunks):
            visit(g, g % nbuf, static=True)

        for g in range(n_chunks - nbuf, n_chunks):
            out_copy(g, g % nbuf).wait()

    grid_spec = pltpu.PrefetchScalarGridSpec(
        num_scalar_prefetch=1,
        grid=(1,),
        in_specs=[pl.BlockSpec(memory_space=pltpu.MemorySpace.HBM)],
        out_specs=pl.BlockSpec(memory_space=pltpu.MemorySpace.HBM),
        scratch_shapes=(
            [pltpu.VMEM((_TG, d), jnp.float32)] * _TNBUF
            + [pltpu.SemaphoreType.DMA] * (2 * _TNBUF)),
    )
    return pl.pallas_call(
        tc_body,
        grid_spec=grid_spec,
        out_shape=jax.ShapeDtypeStruct((n_tc, d), jnp.float32),
    )


def kernel(idx, embedding):
    b, s = idx.shape
    v, d = embedding.shape
    n_tok = b * s
    n_tc = (n_tok * _TC_FRAC_NUM // _TC_FRAC_DEN) // _NW * _NW
    n_sc = n_tok - n_tc

    idx_flat = idx.reshape(n_tok).astype(jnp.int32)
    idx_sc = idx_flat[:n_sc].reshape(_NW, (n_sc // _NW) // _CHUNK, _CHUNK)
    idx_tc = idx_flat[n_sc:]

    sc_full = _make_sc_gather(n_tok, n_sc, d, _CHUNK, _NBUF)(embedding, idx_sc)
    tc_tail = _make_tc_gather(n_tc, v, d)(idx_tc, embedding)
    out = lax.dynamic_update_slice(sc_full, tc_tail, (n_sc, 0))
    return out.reshape(b, s, d)


# speedup vs baseline: 6.1129x; 1.1169x over previous
import functools

import jax
import jax.numpy as jnp
from jax import lax
from jax.experimental import pallas as pl
from jax.experimental.pallas import tpu as pltpu
from jax.experimental.pallas import tpu_sc as plsc

_INFO = plsc.get_sparse_core_info()
_NC = _INFO.num_cores
_NS = _INFO.num_subcores
_NW = _NC * _NS

_NBUF = 3
_CHUNK = 4
_TC_FRAC_NUM, _TC_FRAC_DEN = 11, 64


def _make_sc_gather(n_tok: int, n_sc: int, d: int, chunk: int, nbuf: int):
    b_per_w = n_sc // _NW
    n_chunks = b_per_w // chunk
    assert n_chunks >= 2 * nbuf
    mesh = plsc.VectorSubcoreMesh(core_axis_name="c", subcore_axis_name="s")

    @functools.partial(
        pl.kernel,
        mesh=mesh,
        out_type=jax.ShapeDtypeStruct((n_tok, d), jnp.float32),
        scratch_types=[
            pltpu.VMEM((n_chunks, chunk), jnp.int32),
        ] + [pltpu.VMEM((chunk, d), jnp.float32)] * nbuf
          + [pltpu.SemaphoreType.DMA] * (2 * nbuf),
    )
    def gather_kernel(table_hbm, idx_hbm, out_hbm, idx_v, *rest):
        bufs = rest[:nbuf]
        gsems = rest[nbuf:2 * nbuf]
        wsems = rest[2 * nbuf:]

        wid = lax.axis_index("s") * _NC + lax.axis_index("c")
        base = wid * b_per_w
        pltpu.sync_copy(idx_hbm.at[wid], idx_v)

        def out_rows(g):
            return out_hbm.at[pl.ds(base + g * chunk, chunk)]

        def start_gather(g, b):
            pltpu.async_copy(table_hbm.at[idx_v.at[g]], bufs[b], gsems[b])

        def visit(g, b, static):
            bn = (b + nbuf - 1) % nbuf
            pltpu.make_async_copy(
                table_hbm.at[idx_v.at[g]], bufs[b], gsems[b]).wait()

            def recycle_wait():
                pltpu.make_async_copy(
                    bufs[bn], out_rows(g - 1), wsems[bn]).wait()

            def next_gather():
                start_gather(g + nbuf - 1, bn)

            if static:
                if g >= 1 and g + nbuf - 1 < n_chunks:
                    recycle_wait()
                if g + nbuf - 1 < n_chunks:
                    next_gather()
            else:
                pl.when((g >= 1) & (g + nbuf - 1 < n_chunks))(recycle_wait)
                pl.when(g + nbuf - 1 < n_chunks)(next_gather)

            pltpu.async_copy(bufs[b], out_rows(g), wsems[b])

        for b in range(nbuf - 1):
            start_gather(b, b)

        n_full = (n_chunks // nbuf) * nbuf

        def ring_body(t, carry):
            for b in range(nbuf):
                visit(nbuf * t + b, b, static=False)
            return carry

        lax.fori_loop(0, n_full // nbuf, ring_body, 0)

        for g in range(n_full, n_chunks):
            visit(g, g % nbuf, static=True)

        for g in range(n_chunks - nbuf, n_chunks):
            b = g % nbuf
            pltpu.make_async_copy(bufs[b], out_rows(g), wsems[b]).wait()

    return gather_kernel


_TG = 32
_TNBUF = 8


def _make_tc_gather(n_tc: int, v: int, d: int):
    n_chunks = n_tc // _TG
    assert n_chunks >= 2 * _TNBUF
    nbuf = _TNBUF

    def tc_body(idx_ref, emb_hbm, out_hbm, *rest):
        bufs = rest[:nbuf]
        gsems = rest[nbuf:2 * nbuf]
        wsems = rest[2 * nbuf:]

        def row_copy(g, j, b):
            return pltpu.make_async_copy(
                emb_hbm.at[pl.ds(idx_ref[g * _TG + j], 1), :],
                bufs[b].at[pl.ds(j, 1), :],
                gsems[b])

        def out_copy(g, b):
            return pltpu.make_async_copy(
                bufs[b], out_hbm.at[pl.ds(g * _TG, _TG)], wsems[b])

        def start_gather(g, b):
            for j in range(_TG):
                row_copy(g, j, b).start()

        def visit(g, b, static):
            bn = (b + nbuf - 1) % nbuf
            for j in range(_TG):
                row_copy(g, j, b).wait()

            def recycle_wait():
                out_copy(g - 1, bn).wait()

            def next_gather():
                start_gather(g + nbuf - 1, bn)

            if static:
                if g >= 1 and g + nbuf - 1 < n_chunks:
                    recycle_wait()
                if g + nbuf - 1 < n_chunks:
                    next_gather()
            else:
                pl.when((g >= 1) & (g + nbuf - 1 < n_chunks))(recycle_wait)
                pl.when(g + nbuf - 1 < n_chunks)(next_gather)

            out_copy(g, b).start()

        for b in range(nbuf - 1):
            start_gather(b, b)

        n_full = (n_chunks // nbuf) * nbuf

        def ring_body(t, carry):
            for b in range(nbuf):
                visit(nbuf * t + b, b, static=False)
            return carry

        lax.fori_loop(0, n_full // nbuf, ring_body, 0)

        for g in range(n_full, n_chunks):
            visit(g, g % nbuf, static=True)

        for g in range(n_chunks - nbuf, n_chunks):
            out_copy(g, g % nbuf).wait()

    grid_spec = pltpu.PrefetchScalarGridSpec(
        num_scalar_prefetch=1,
        grid=(1,),
        in_specs=[pl.BlockSpec(memory_space=pltpu.MemorySpace.HBM)],
        out_specs=pl.BlockSpec(memory_space=pltpu.MemorySpace.HBM),
        scratch_shapes=(
            [pltpu.VMEM((_TG, d), jnp.float32)] * _TNBUF
            + [pltpu.SemaphoreType.DMA] * (2 * _TNBUF)),
    )
    return pl.pallas_call(
        tc_body,
        grid_spec=grid_spec,
        out_shape=jax.ShapeDtypeStruct((n_tc, d), jnp.float32),
    )


def kernel(idx, embedding):
    b, s = idx.shape
    v, d = embedding.shape
    n_tok = b * s
    n_tc = (n_tok * _TC_FRAC_NUM // _TC_FRAC_DEN) // _NW * _NW
    n_sc = n_tok - n_tc

    idx_flat = idx.reshape(n_tok).astype(jnp.int32)
    idx_sc = idx_flat[:n_sc].reshape(_NW, (n_sc // _NW) // _CHUNK, _CHUNK)
    idx_tc = idx_flat[n_sc:]

    sc_full = _make_sc_gather(n_tok, n_sc, d, _CHUNK, _NBUF)(embedding, idx_sc)
    tc_tail = _make_tc_gather(n_tc, v, d)(idx_tc, embedding)
    out = lax.dynamic_update_slice(sc_full, tc_tail, (n_sc, 0))
    return out.reshape(b, s, d)


# speedup vs baseline: 6.8773x; 1.1251x over previous
import functools

import jax
import jax.numpy as jnp
from jax import lax
from jax.experimental import pallas as pl
from jax.experimental.pallas import tpu as pltpu
from jax.experimental.pallas import tpu_sc as plsc

_INFO = plsc.get_sparse_core_info()
_NC = _INFO.num_cores
_NS = _INFO.num_subcores
_NW = _NC * _NS

_NBUF = 3
_CHUNK = 4


def _make_gather(n_tok: int, d: int, chunk: int, nbuf: int):
    b_per_w = n_tok // _NW
    n_chunks = b_per_w // chunk
    assert n_chunks >= 2 * nbuf
    mesh = plsc.VectorSubcoreMesh(core_axis_name="c", subcore_axis_name="s")

    @functools.partial(
        pl.kernel,
        mesh=mesh,
        out_type=jax.ShapeDtypeStruct((n_tok, d), jnp.float32),
        scratch_types=[
            pltpu.VMEM((n_chunks, chunk), jnp.int32),
        ] + [pltpu.VMEM((chunk, d), jnp.float32)] * nbuf
          + [pltpu.SemaphoreType.DMA] * (2 * nbuf),
    )
    def gather_kernel(table_hbm, idx_hbm, out_hbm, idx_v, *rest):
        bufs = rest[:nbuf]
        gsems = rest[nbuf:2 * nbuf]
        wsems = rest[2 * nbuf:]

        wid = lax.axis_index("s") * _NC + lax.axis_index("c")
        base = wid * b_per_w
        pltpu.sync_copy(idx_hbm.at[wid], idx_v)

        def out_rows(g):
            return out_hbm.at[pl.ds(base + g * chunk, chunk)]

        def start_gather(g, b):
            pltpu.async_copy(table_hbm.at[idx_v.at[g]], bufs[b], gsems[b])

        def visit(g, b, static):
            bn = (b + nbuf - 1) % nbuf
            pltpu.make_async_copy(
                table_hbm.at[idx_v.at[g]], bufs[b], gsems[b]).wait()

            def recycle_wait():
                pltpu.make_async_copy(
                    bufs[bn], out_rows(g - 1), wsems[bn]).wait()

            def next_gather():
                start_gather(g + nbuf - 1, bn)

            if static:
                if g >= 1 and g + nbuf - 1 < n_chunks:
                    recycle_wait()
                if g + nbuf - 1 < n_chunks:
                    next_gather()
            else:
                pl.when((g >= 1) & (g + nbuf - 1 < n_chunks))(recycle_wait)
                pl.when(g + nbuf - 1 < n_chunks)(next_gather)

            pltpu.async_copy(bufs[b], out_rows(g), wsems[b])

        for b in range(nbuf - 1):
            start_gather(b, b)

        n_full = (n_chunks // nbuf) * nbuf

        def ring_body(t, carry):
            for b in range(nbuf):
                visit(nbuf * t + b, b, static=False)
            return carry

        lax.fori_loop(0, n_full // nbuf, ring_body, 0)

        for g in range(n_full, n_chunks):
            visit(g, g % nbuf, static=True)

        for g in range(n_chunks - nbuf, n_chunks):
            b = g % nbuf
            pltpu.make_async_copy(bufs[b], out_rows(g), wsems[b]).wait()

    return gather_kernel


def kernel(idx, embedding):
    b, s = idx.shape
    v, d = embedding.shape
    n_tok = b * s
    idx32 = idx.reshape(_NW, (n_tok // _NW) // _CHUNK, _CHUNK).astype(jnp.int32)
    out = _make_gather(n_tok, d, _CHUNK, _NBUF)(embedding, idx32)
    return out.reshape(b, s, d)
